# baseline (speedup 1.0000x reference)
import functools

import jax
import jax.numpy as jnp
from jax import lax
from jax.experimental import pallas as pl
from jax.experimental.pallas import tpu as pltpu
from jax.experimental.pallas import tpu_sc as plsc

N = 10000
IN_DIM = 128
OUT_DIM = 16
E = 320000

NC = 2
NS = 16
L = 16
NW = NC * NS

N_PAD = 10240
NPK = N_PAD * OUT_DIM // 128
RPT = N_PAD // NS
CHUNK = 80
K = 125
EPT = K * CHUNK

NB = 5
KB = K // NB

_mesh = plsc.VectorSubcoreMesh(core_axis_name="c", subcore_axis_name="s")


def _rsqrt16(d):
    i = plsc.bitcast(d, jnp.int32)
    i = jnp.int32(0x5F3759DF) - lax.shift_right_logical(i, 1)
    y = plsc.bitcast(i, jnp.float32)
    for _ in range(3):
        y = y * (1.5 - 0.5 * d * y * y)
    return y


@functools.partial(
    pl.kernel,
    out_type=jax.ShapeDtypeStruct((NC, N_PAD), jnp.float32),
    mesh=_mesh,
    scratch_types=[
        pltpu.VMEM((EPT,), jnp.int32),
        pltpu.VMEM((CHUNK,), jnp.float32),
        pltpu.VMEM((RPT,), jnp.float32),
        pltpu.VMEM_SHARED((N_PAD,), jnp.float32),
    ] + [pltpu.SemaphoreType.DMA] * NB,
    compiler_params=pltpu.CompilerParams(use_tc_tiling_on_sc=False,
                                         needs_layout_passes=False),
)
def _deg_kernel(ei_hbm, degp_hbm, didx_v, ones_v, zb_v, deg_sh, *ssem):
    c = lax.axis_index("c")
    s = lax.axis_index("s")
    wid = s * NC + c
    row0 = s * RPT
    one = jnp.ones((L,), jnp.float32)
    zero = jnp.zeros((L,), jnp.float32)
    for i in range(CHUNK // L):
        ones_v[pl.ds(i * L, L)] = one
    for i in range(RPT // L):
        zb_v[pl.ds(i * L, L)] = zero
    pltpu.sync_copy(zb_v, deg_sh.at[pl.ds(row0, RPT)])
    pltpu.sync_copy(ei_hbm.at[1].at[pl.ds(wid * EPT, EPT)], didx_v)
    plsc.subcore_barrier()

    for b in range(NB):
        pltpu.async_copy(ones_v, deg_sh.at[didx_v.at[pl.ds(b * CHUNK, CHUNK)]],
                         ssem[b], add=True)

    def dbody(t, carry):
        for b in range(NB):
            j = t * NB + b
            pltpu.make_async_copy(
                ones_v, deg_sh.at[didx_v.at[pl.ds(j * CHUNK, CHUNK)]],
                ssem[b]).wait()
            pltpu.async_copy(
                ones_v, deg_sh.at[didx_v.at[pl.ds((j + NB) * CHUNK, CHUNK)]],
                ssem[b], add=True)
        return carry

    lax.fori_loop(0, KB - 1, dbody, 0)
    for b in range(NB):
        j = (KB - 1) * NB + b
        pltpu.make_async_copy(
            ones_v, deg_sh.at[didx_v.at[pl.ds(j * CHUNK, CHUNK)]],
            ssem[b]).wait()
    plsc.subcore_barrier()
    pltpu.sync_copy(deg_sh.at[pl.ds(row0, RPT)],
                    degp_hbm.at[c].at[pl.ds(row0, RPT)])


@functools.partial(
    pl.kernel,
    out_type=jax.ShapeDtypeStruct((NC, N_PAD, OUT_DIM), jnp.float32),
    mesh=_mesh,
    scratch_types=[
        pltpu.VMEM((EPT,), jnp.int32),
        pltpu.VMEM((EPT,), jnp.int32),
        pltpu.VMEM((RPT,), jnp.float32),
        pltpu.VMEM((RPT,), jnp.float32),
        pltpu.VMEM((RPT, OUT_DIM), jnp.float32),
        pltpu.VMEM((NB, CHUNK, OUT_DIM), jnp.float32),
        pltpu.VMEM_SHARED((N_PAD, OUT_DIM), jnp.float32),
        pltpu.VMEM_SHARED((N_PAD, OUT_DIM), jnp.float32),
    ] + [pltpu.SemaphoreType.DMA] * (2 * NB),
    compiler_params=pltpu.CompilerParams(use_tc_tiling_on_sc=False,
                                         needs_layout_passes=False),
)
def _agg_kernel(ei_hbm, h_hbm, degp_hbm, accp_hbm,
                sidx_v, didx_v, degv, dinvv, hv, rows_v,
                g_sh, acc_sh, *sems):
    gsem, ssem = sems[:NB], sems[NB:]
    c = lax.axis_index("c")
    s = lax.axis_index("s")
    wid = s * NC + c
    row0 = s * RPT
    zero = jnp.zeros((L,), jnp.float32)

    for i in range(CHUNK):
        rows_v[0, i, :] = zero
    for t in range(RPT // CHUNK):
        pltpu.sync_copy(rows_v.at[0], acc_sh.at[pl.ds(row0 + t * CHUNK, CHUNK)])
    pltpu.sync_copy(ei_hbm.at[0].at[pl.ds(wid * EPT, EPT)], sidx_v)
    pltpu.sync_copy(ei_hbm.at[1].at[pl.ds(wid * EPT, EPT)], didx_v)
    pltpu.sync_copy(h_hbm.at[pl.ds(row0, RPT)], hv)
    pltpu.sync_copy(degp_hbm.at[0].at[pl.ds(row0, RPT)], degv)
    pltpu.sync_copy(degp_hbm.at[1].at[pl.ds(row0, RPT)], dinvv)

    def vbody(i, carry):
        d = degv[pl.ds(i * L, L)] + dinvv[pl.ds(i * L, L)] + 1.0
        dinvv[pl.ds(i * L, L)] = _rsqrt16(d)
        return carry

    lax.fori_loop(0, RPT // L, vbody, 0)

    def gbody(i, carry):
        dv = dinvv[pl.ds(i * L, L)]
        for bl in range(L):
            r = i * L + bl
            hv[r, :] = hv[r, :] * dv[bl]
        return carry

    lax.fori_loop(0, RPT // L, gbody, 0)
    pltpu.sync_copy(hv, g_sh.at[pl.ds(row0, RPT)])
    plsc.subcore_barrier()

    for b in range(NB):
        pltpu.async_copy(g_sh.at[sidx_v.at[pl.ds(b * CHUNK, CHUNK)]],
                         rows_v.at[b], gsem[b])

    def abody(t, carry):
        j0 = t * NB
        for b in range(NB):
            j = j0 + b
            pltpu.make_async_copy(
                g_sh.at[sidx_v.at[pl.ds(j * CHUNK, CHUNK)]], rows_v.at[b],
                gsem[b]).wait()
            pltpu.async_copy(
                rows_v.at[b], acc_sh.at[didx_v.at[pl.ds(j * CHUNK, CHUNK)]],
                ssem[b], add=True)
        for b in range(NB):
            j = j0 + b
            pltpu.make_async_copy(
                rows_v.at[b], acc_sh.at[didx_v.at[pl.ds(j * CHUNK, CHUNK)]],
                ssem[b]).wait()
            pltpu.async_copy(
                g_sh.at[sidx_v.at[pl.ds((j + NB) * CHUNK, CHUNK)]],
                rows_v.at[b], gsem[b])
        return carry

    lax.fori_loop(0, KB - 1, abody, 0)
    for b in range(NB):
        j = (KB - 1) * NB + b
        pltpu.make_async_copy(
            g_sh.at[sidx_v.at[pl.ds(j * CHUNK, CHUNK)]], rows_v.at[b],
            gsem[b]).wait()
        pltpu.async_copy(
            rows_v.at[b], acc_sh.at[didx_v.at[pl.ds(j * CHUNK, CHUNK)]],
            ssem[b], add=True)
    for b in range(NB):
        j = (KB - 1) * NB + b
        pltpu.make_async_copy(
            rows_v.at[b], acc_sh.at[didx_v.at[pl.ds(j * CHUNK, CHUNK)]],
            ssem[b]).wait()
    plsc.subcore_barrier()
    pltpu.sync_copy(acc_sh.at[pl.ds(row0, RPT)],
                    accp_hbm.at[c].at[pl.ds(row0, RPT)])


RPW = N_PAD // NW


@functools.partial(
    pl.kernel,
    out_type=jax.ShapeDtypeStruct((N_PAD, OUT_DIM), jnp.float32),
    mesh=_mesh,
    scratch_types=[
        pltpu.VMEM((RPW, OUT_DIM), jnp.float32),
        pltpu.VMEM((RPW, OUT_DIM), jnp.float32),
        pltpu.VMEM((RPW, OUT_DIM), jnp.float32),
        pltpu.VMEM((RPW,), jnp.float32),
        pltpu.VMEM((RPW,), jnp.float32),
        pltpu.VMEM((OUT_DIM,), jnp.float32),
    ],
    compiler_params=pltpu.CompilerParams(use_tc_tiling_on_sc=False,
                                         needs_layout_passes=False),
)
def _combine_kernel(accp_hbm, h_hbm, degp_hbm, b_hbm, out_hbm,
                    a0v, a1v, hv, degv, degv1, bv):
    c = lax.axis_index("c")
    s = lax.axis_index("s")
    wid = s * NC + c
    r0 = wid * RPW
    pltpu.sync_copy(accp_hbm.at[0].at[pl.ds(r0, RPW)], a0v)
    pltpu.sync_copy(accp_hbm.at[1].at[pl.ds(r0, RPW)], a1v)
    pltpu.sync_copy(h_hbm.at[pl.ds(r0, RPW)], hv)
    pltpu.sync_copy(degp_hbm.at[0].at[pl.ds(r0, RPW)], degv)
    pltpu.sync_copy(degp_hbm.at[1].at[pl.ds(r0, RPW)], degv1)
    pltpu.sync_copy(b_hbm, bv)
    bb = bv[...]

    def body(i, carry):
        dv = _rsqrt16(degv[pl.ds(i * L, L)] + degv1[pl.ds(i * L, L)] + 1.0)
        for bl in range(L):
            r = i * L + bl
            hv[r, :] = (a0v[r, :] + a1v[r, :] + hv[r, :] * dv[bl]) * dv[bl] + bb
        return carry

    lax.fori_loop(0, RPW // L, body, 0)
    pltpu.sync_copy(hv, out_hbm.at[pl.ds(r0, RPW)])


def _linear_body(x_ref, w_ref, h_ref):
    h = lax.dot_general(x_ref[...], w_ref[...],
                        (((1,), (1,)), ((), ())),
                        preferred_element_type=jnp.float32)
    h_ref[:N, :] = h
    h_ref[N:, :] = jnp.zeros((N_PAD - N, OUT_DIM), jnp.float32)


def _unpack_body(p_ref, o_ref):
    for u in range(8):
        o_ref[:, u, :] = p_ref[:N // 8, u * OUT_DIM:(u + 1) * OUT_DIM]


def kernel(x, edge_index, W, b):
    ei = lax.optimization_barrier(edge_index.astype(jnp.int32))
    degp = _deg_kernel(ei)
    h = pl.pallas_call(
        _linear_body,
        out_shape=jax.ShapeDtypeStruct((N_PAD, OUT_DIM), jnp.float32),
    )(x, W)
    accp = _agg_kernel(ei, h, degp)
    out_full = _combine_kernel(accp, h, degp, b)
    out_pk = out_full.reshape(NPK, 128)
    out3 = pl.pallas_call(
        _unpack_body,
        out_shape=jax.ShapeDtypeStruct((N // 8, 8, OUT_DIM), jnp.float32),
    )(out_pk)
    return out3.reshape(N, OUT_DIM)


# speedup vs baseline: 1.0963x; 1.0963x over previous
import functools

import jax
import jax.numpy as jnp
from jax import lax
from jax.experimental import pallas as pl
from jax.experimental.pallas import tpu as pltpu
from jax.experimental.pallas import tpu_sc as plsc

N = 10000
IN_DIM = 128
OUT_DIM = 16
E = 320000

NC = 2
NS = 16
L = 16
NW = NC * NS

N_PAD = 10240
NPK = N_PAD * OUT_DIM // 128
RPT = N_PAD // NS
CHUNK = 80
K = 125
EPT = K * CHUNK

NB = 5
KB = K // NB

_mesh = plsc.VectorSubcoreMesh(core_axis_name="c", subcore_axis_name="s")


def _rsqrt16(d):
    i = plsc.bitcast(d, jnp.int32)
    i = jnp.int32(0x5F3759DF) - lax.shift_right_logical(i, 1)
    y = plsc.bitcast(i, jnp.float32)
    for _ in range(3):
        y = y * (1.5 - 0.5 * d * y * y)
    return y


@functools.partial(
    pl.kernel,
    out_type=jax.ShapeDtypeStruct((NC, N_PAD), jnp.float32),
    mesh=_mesh,
    scratch_types=[
        pltpu.VMEM((EPT,), jnp.int32),
        pltpu.VMEM((CHUNK,), jnp.float32),
        pltpu.VMEM((RPT,), jnp.float32),
        pltpu.VMEM_SHARED((N_PAD,), jnp.float32),
    ] + [pltpu.SemaphoreType.DMA] * NB,
    compiler_params=pltpu.CompilerParams(use_tc_tiling_on_sc=False,
                                         needs_layout_passes=False),
)
def _deg_kernel(ei_hbm, degp_hbm, didx_v, ones_v, zb_v, deg_sh, *ssem):
    c = lax.axis_index("c")
    s = lax.axis_index("s")
    wid = s * NC + c
    row0 = s * RPT
    one = jnp.ones((L,), jnp.float32)
    zero = jnp.zeros((L,), jnp.float32)
    for i in range(CHUNK // L):
        ones_v[pl.ds(i * L, L)] = one
    for i in range(RPT // L):
        zb_v[pl.ds(i * L, L)] = zero
    pltpu.sync_copy(zb_v, deg_sh.at[pl.ds(row0, RPT)])
    pltpu.sync_copy(ei_hbm.at[1].at[pl.ds(wid * EPT, EPT)], didx_v)
    plsc.subcore_barrier()

    for b in range(NB):
        pltpu.async_copy(ones_v, deg_sh.at[didx_v.at[pl.ds(b * CHUNK, CHUNK)]],
                         ssem[b], add=True)

    def dbody(t, carry):
        for b in range(NB):
            j = t * NB + b
            pltpu.make_async_copy(
                ones_v, deg_sh.at[didx_v.at[pl.ds(j * CHUNK, CHUNK)]],
                ssem[b]).wait()
            pltpu.async_copy(
                ones_v, deg_sh.at[didx_v.at[pl.ds((j + NB) * CHUNK, CHUNK)]],
                ssem[b], add=True)
        return carry

    lax.fori_loop(0, KB - 1, dbody, 0)
    for b in range(NB):
        j = (KB - 1) * NB + b
        pltpu.make_async_copy(
            ones_v, deg_sh.at[didx_v.at[pl.ds(j * CHUNK, CHUNK)]],
            ssem[b]).wait()
    plsc.subcore_barrier()
    pltpu.sync_copy(deg_sh.at[pl.ds(row0, RPT)],
                    degp_hbm.at[c].at[pl.ds(row0, RPT)])


@functools.partial(
    pl.kernel,
    out_type=jax.ShapeDtypeStruct((NC, N_PAD, OUT_DIM), jnp.float32),
    mesh=_mesh,
    scratch_types=[
        pltpu.VMEM((EPT,), jnp.int32),
        pltpu.VMEM((EPT,), jnp.int32),
        pltpu.VMEM((RPT,), jnp.float32),
        pltpu.VMEM((RPT,), jnp.float32),
        pltpu.VMEM((RPT, OUT_DIM), jnp.float32),
        pltpu.VMEM((NB, CHUNK, OUT_DIM), jnp.float32),
        pltpu.VMEM_SHARED((N_PAD, OUT_DIM), jnp.float32),
        pltpu.VMEM_SHARED((N_PAD, OUT_DIM), jnp.float32),
    ] + [pltpu.SemaphoreType.DMA] * (2 * NB),
    compiler_params=pltpu.CompilerParams(use_tc_tiling_on_sc=False,
                                         needs_layout_passes=False),
)
def _agg_kernel(ei_hbm, h_hbm, degp_hbm, accp_hbm,
                sidx_v, didx_v, degv, dinvv, hv, rows_v,
                g_sh, acc_sh, *sems):
    gsem, ssem = sems[:NB], sems[NB:]
    c = lax.axis_index("c")
    s = lax.axis_index("s")
    wid = s * NC + c
    row0 = s * RPT
    zero = jnp.zeros((L,), jnp.float32)

    for i in range(CHUNK):
        rows_v[0, i, :] = zero
    for t in range(RPT // CHUNK):
        pltpu.sync_copy(rows_v.at[0], acc_sh.at[pl.ds(row0 + t * CHUNK, CHUNK)])
    pltpu.sync_copy(ei_hbm.at[0].at[pl.ds(wid * EPT, EPT)], sidx_v)
    pltpu.sync_copy(ei_hbm.at[1].at[pl.ds(wid * EPT, EPT)], didx_v)
    pltpu.sync_copy(h_hbm.at[pl.ds(row0, RPT)], hv)
    pltpu.sync_copy(degp_hbm.at[0].at[pl.ds(row0, RPT)], degv)
    pltpu.sync_copy(degp_hbm.at[1].at[pl.ds(row0, RPT)], dinvv)

    def vbody(i, carry):
        d = degv[pl.ds(i * L, L)] + dinvv[pl.ds(i * L, L)] + 1.0
        dinvv[pl.ds(i * L, L)] = _rsqrt16(d)
        return carry

    lax.fori_loop(0, RPT // L, vbody, 0)

    def gbody(i, carry):
        dv = dinvv[pl.ds(i * L, L)]
        for bl in range(L):
            r = i * L + bl
            hv[r, :] = hv[r, :] * dv[bl]
        return carry

    lax.fori_loop(0, RPT // L, gbody, 0)
    pltpu.sync_copy(hv, g_sh.at[pl.ds(row0, RPT)])
    plsc.subcore_barrier()

    for b in range(NB):
        pltpu.async_copy(g_sh.at[sidx_v.at[pl.ds(b * CHUNK, CHUNK)]],
                         rows_v.at[b], gsem[b])

    def abody(t, carry):
        j0 = t * NB
        for b in range(NB):
            j = j0 + b
            pltpu.make_async_copy(
                g_sh.at[sidx_v.at[pl.ds(j * CHUNK, CHUNK)]], rows_v.at[b],
                gsem[b]).wait()
            pltpu.async_copy(
                rows_v.at[b], acc_sh.at[didx_v.at[pl.ds(j * CHUNK, CHUNK)]],
                ssem[b], add=True)
        for b in range(NB):
            j = j0 + b
            pltpu.make_async_copy(
                rows_v.at[b], acc_sh.at[didx_v.at[pl.ds(j * CHUNK, CHUNK)]],
                ssem[b]).wait()
            pltpu.async_copy(
                g_sh.at[sidx_v.at[pl.ds((j + NB) * CHUNK, CHUNK)]],
                rows_v.at[b], gsem[b])
        return carry

    lax.fori_loop(0, KB - 1, abody, 0)
    for b in range(NB):
        j = (KB - 1) * NB + b
        pltpu.make_async_copy(
            g_sh.at[sidx_v.at[pl.ds(j * CHUNK, CHUNK)]], rows_v.at[b],
            gsem[b]).wait()
        pltpu.async_copy(
            rows_v.at[b], acc_sh.at[didx_v.at[pl.ds(j * CHUNK, CHUNK)]],
            ssem[b], add=True)
    for b in range(NB):
        j = (KB - 1) * NB + b
        pltpu.make_async_copy(
            rows_v.at[b], acc_sh.at[didx_v.at[pl.ds(j * CHUNK, CHUNK)]],
            ssem[b]).wait()
    plsc.subcore_barrier()
    pltpu.sync_copy(acc_sh.at[pl.ds(row0, RPT)],
                    accp_hbm.at[c].at[pl.ds(row0, RPT)])


RPW = N_PAD // NW


@functools.partial(
    pl.kernel,
    out_type=jax.ShapeDtypeStruct((N_PAD, OUT_DIM), jnp.float32),
    mesh=_mesh,
    scratch_types=[
        pltpu.VMEM((RPW, OUT_DIM), jnp.float32),
        pltpu.VMEM((RPW, OUT_DIM), jnp.float32),
        pltpu.VMEM((RPW, OUT_DIM), jnp.float32),
        pltpu.VMEM((RPW,), jnp.float32),
        pltpu.VMEM((RPW,), jnp.float32),
        pltpu.VMEM((OUT_DIM,), jnp.float32),
    ],
    compiler_params=pltpu.CompilerParams(use_tc_tiling_on_sc=False,
                                         needs_layout_passes=False),
)
def _combine_kernel(accp_hbm, h_hbm, degp_hbm, b_hbm, out_hbm,
                    a0v, a1v, hv, degv, degv1, bv):
    c = lax.axis_index("c")
    s = lax.axis_index("s")
    wid = s * NC + c
    r0 = wid * RPW
    pltpu.sync_copy(accp_hbm.at[0].at[pl.ds(r0, RPW)], a0v)
    pltpu.sync_copy(accp_hbm.at[1].at[pl.ds(r0, RPW)], a1v)
    pltpu.sync_copy(h_hbm.at[pl.ds(r0, RPW)], hv)
    pltpu.sync_copy(degp_hbm.at[0].at[pl.ds(r0, RPW)], degv)
    pltpu.sync_copy(degp_hbm.at[1].at[pl.ds(r0, RPW)], degv1)
    pltpu.sync_copy(b_hbm, bv)
    bb = bv[...]

    def body(i, carry):
        dv = _rsqrt16(degv[pl.ds(i * L, L)] + degv1[pl.ds(i * L, L)] + 1.0)
        for bl in range(L):
            r = i * L + bl
            hv[r, :] = (a0v[r, :] + a1v[r, :] + hv[r, :] * dv[bl]) * dv[bl] + bb
        return carry

    lax.fori_loop(0, RPW // L, body, 0)
    pltpu.sync_copy(hv, out_hbm.at[pl.ds(r0, RPW)])


def _linear_body(x_ref, w_ref, h_ref):
    h = lax.dot_general(x_ref[...], w_ref[...],
                        (((1,), (1,)), ((), ())),
                        preferred_element_type=jnp.float32)
    h_ref[:N, :] = h
    h_ref[N:, :] = jnp.zeros((N_PAD - N, OUT_DIM), jnp.float32)


def kernel(x, edge_index, W, b):
    ei = lax.optimization_barrier(edge_index.astype(jnp.int32))
    degp = _deg_kernel(ei)
    h = pl.pallas_call(
        _linear_body,
        out_shape=jax.ShapeDtypeStruct((N_PAD, OUT_DIM), jnp.float32),
    )(x, W)
    accp = _agg_kernel(ei, h, degp)
    out_full = _combine_kernel(accp, h, degp, b)
    return out_full[:N]
